# Initial kernel scaffold; baseline (speedup 1.0000x reference)
import jax
import jax.numpy as jnp
from jax.experimental import pallas as pl


def kernel(x, table):
    raise NotImplementedError("write your pallas kernel here")


# speedup vs baseline: 2.3206x; 2.3206x over previous
import functools

import jax
import jax.numpy as jnp
from jax import lax
from jax.experimental import pallas as pl
from jax.experimental.pallas import tpu as pltpu
from jax.experimental.pallas import tpu_sc as plsc

_NUM_BUCKETS = 1000
_EMBED_DIM = 128
_BATCH = 16384
_DIV = 100.0 + 1e-8

_NC = 2
_NS = 16
_L = 16
_NW = _NC * _NS
_BPW = _BATCH // _NW
_CHUNK = 128
_NCHUNK = _BPW // _CHUNK


def _body(x_hbm, table_hbm, out_hbm, x_v, idx_v, rows_v, sem):
    wid = lax.axis_index("s") * _NC + lax.axis_index("c")
    base = wid * _BPW
    pltpu.sync_copy(x_hbm.at[pl.ds(base, _BPW)], x_v)

    copies = []
    for j in range(_NCHUNK):
        for i in range(_CHUNK // _L):
            xv = x_v[pl.ds(j * _CHUNK + i * _L, _L)]
            y = (xv / jnp.float32(_DIV)) * jnp.float32(_NUM_BUCKETS)
            idx = jnp.clip(y.astype(jnp.int32), 0, _NUM_BUCKETS - 1)
            idx_v[j, pl.ds(i * _L, _L)] = idx
        copies.append(
            pltpu.async_copy(
                table_hbm.at[idx_v.at[j]],
                rows_v.at[pl.ds(j * _CHUNK, _CHUNK)],
                sem,
            )
        )
    for c in copies:
        c.wait()
    pltpu.sync_copy(rows_v, out_hbm.at[pl.ds(base, _BPW)])


_sc_lookup = functools.partial(
    pl.kernel,
    out_type=jax.ShapeDtypeStruct((_BATCH, _EMBED_DIM), jnp.float32),
    mesh=plsc.VectorSubcoreMesh(core_axis_name="c", subcore_axis_name="s"),
    scratch_types=[
        pltpu.VMEM((_BPW,), jnp.float32),
        pltpu.VMEM((_NCHUNK, _CHUNK), jnp.int32),
        pltpu.VMEM((_BPW, _EMBED_DIM), jnp.float32),
        pltpu.SemaphoreType.DMA,
    ],
)(_body)


def kernel(x, table):
    return _sc_lookup(x, table)
